# Initial kernel scaffold; baseline (speedup 1.0000x reference)
import jax
import jax.numpy as jnp
from jax.experimental import pallas as pl


def kernel(x, router_W, router_b, expert_W1, expert_b1, expert_W2, expert_b2, micro_router_W, micro_router_b, micro_W1, micro_b1, micro_W2, micro_b2, micro_ln_g, micro_ln_b, norm_g, norm_b):
    raise NotImplementedError("write your pallas kernel here")


# speedup vs baseline: 2.8504x; 2.8504x over previous
import functools

import jax
import jax.numpy as jnp
from jax.experimental import pallas as pl
from jax.experimental.pallas import tpu as pltpu

DIM = 768
NUM_EXPERTS = 8
NUM_MICROS = 16
TOP_K = 2
TOP_K_MICROS = 8
EXPERT_DIM = 1536
MICRO_HID = DIM // 2
SEQ = 2048
TILE = 512
NUM_TILES = SEQ // TILE
EPS = 1e-5


def _gelu(v):
    return 0.5 * v * (1.0 + jax.lax.erf(v * 0.7071067811865476))


def _layer_norm(v, g, b):
    mu = jnp.mean(v, axis=-1, keepdims=True)
    var = jnp.mean((v - mu) ** 2, axis=-1, keepdims=True)
    return (v - mu) * jax.lax.rsqrt(var + EPS) * g + b


def _topk_mask_combine(probs, k):
    n = probs.shape[-1]
    work = probs
    thr = None
    sel_sum = jnp.zeros(probs.shape[:-1] + (1,), probs.dtype)
    for _ in range(k):
        thr = jnp.max(work, axis=-1, keepdims=True)
        sel_sum = sel_sum + thr
        work = jnp.where(work >= thr, -jnp.inf, work)
    mask = probs >= thr
    return jnp.where(mask, probs, 0.0) / (sel_sum + 1e-8)


def _expert_kernel(x_ref, rw_ref, rb_ref, w1_ref, b1_ref, w2_ref, b2_ref,
                   out_ref):
    e = pl.program_id(0)
    t = pl.program_id(1)
    xt = x_ref[pl.ds(t * TILE, TILE), :]
    logits = jnp.dot(xt, rw_ref[...], preferred_element_type=jnp.float32)
    logits = logits + rb_ref[...]
    probs = jax.nn.softmax(logits, axis=-1)
    combine = _topk_mask_combine(probs, TOP_K)
    lane = jax.lax.broadcasted_iota(jnp.int32, combine.shape, 1)
    col = jnp.sum(jnp.where(lane == e, combine, 0.0), axis=-1, keepdims=True)

    h = jnp.dot(xt, w1_ref[0], preferred_element_type=jnp.float32)
    h = _gelu(h + b1_ref[0])
    eo = jnp.dot(h, w2_ref[0], preferred_element_type=jnp.float32)
    eo = (eo + b2_ref[0]) * col

    @pl.when(e == 0)
    def _init():
        out_ref[pl.ds(t * TILE, TILE), :] = eo

    @pl.when(e > 0)
    def _acc():
        out_ref[pl.ds(t * TILE, TILE), :] += eo


def _micro_kernel(eo_ref, rw_ref, rb_ref, w1_ref, b1_ref, w2_ref, b2_ref,
                  lng_ref, lnb_ref, ng_ref, nb_ref, out_ref, acc_ref):
    m = pl.program_id(0)
    t = pl.program_id(1)
    xt = eo_ref[pl.ds(t * TILE, TILE), :]
    logits = jnp.dot(xt, rw_ref[...], preferred_element_type=jnp.float32)
    logits = logits + rb_ref[...]
    probs = jax.nn.softmax(logits, axis=-1)
    mcombine = _topk_mask_combine(probs, TOP_K_MICROS)
    lane = jax.lax.broadcasted_iota(jnp.int32, mcombine.shape, 1)
    col = jnp.sum(jnp.where(lane == m, mcombine, 0.0), axis=-1, keepdims=True)

    mh = jnp.dot(xt, w1_ref[0], preferred_element_type=jnp.float32)
    mh = _gelu(mh + b1_ref[0])
    mf = jnp.dot(mh, w2_ref[0], preferred_element_type=jnp.float32)
    pre = xt + mf + b2_ref[0]
    mo = _layer_norm(pre, lng_ref[0], lnb_ref[0]) * col

    @pl.when(m == 0)
    def _init():
        acc_ref[pl.ds(t * TILE, TILE), :] = mo

    @pl.when(m > 0)
    def _acc():
        acc_ref[pl.ds(t * TILE, TILE), :] += mo

    @pl.when(m == NUM_MICROS - 1)
    def _final():
        combined = xt + 0.1 * acc_ref[pl.ds(t * TILE, TILE), :]
        out_ref[pl.ds(t * TILE, TILE), :] = _layer_norm(
            combined, ng_ref[...], nb_ref[...])


def _resident(shape):
    return pl.BlockSpec(shape, lambda *_: tuple(0 for _ in shape))


def _per_e(shape):
    return pl.BlockSpec(shape, lambda e, t: (e,) + tuple(0 for _ in shape[1:]))


@jax.jit
def kernel(x, router_W, router_b, expert_W1, expert_b1, expert_W2, expert_b2,
           micro_router_W, micro_router_b, micro_W1, micro_b1, micro_W2,
           micro_b2, micro_ln_g, micro_ln_b, norm_g, norm_b):
    B, S, D = x.shape
    xf = x.reshape(S, D)

    expert_output = pl.pallas_call(
        _expert_kernel,
        grid=(NUM_EXPERTS, NUM_TILES),
        in_specs=[
            _resident((S, D)),
            _resident((D, NUM_EXPERTS)),
            _resident((1, NUM_EXPERTS)),
            _per_e((1, DIM, EXPERT_DIM)),
            _per_e((1, 1, EXPERT_DIM)),
            _per_e((1, EXPERT_DIM, DIM)),
            _per_e((1, 1, DIM)),
        ],
        out_specs=_resident((S, D)),
        out_shape=jax.ShapeDtypeStruct((S, D), jnp.float32),
        compiler_params=pltpu.CompilerParams(
            dimension_semantics=("arbitrary", "arbitrary"),
        ),
    )(xf, router_W, router_b.reshape(1, -1), expert_W1,
      expert_b1.reshape(NUM_EXPERTS, 1, EXPERT_DIM), expert_W2,
      expert_b2.reshape(NUM_EXPERTS, 1, DIM))

    out = pl.pallas_call(
        _micro_kernel,
        grid=(NUM_MICROS, NUM_TILES),
        in_specs=[
            _resident((S, D)),
            _resident((D, NUM_MICROS)),
            _resident((1, NUM_MICROS)),
            _per_e((1, DIM, MICRO_HID)),
            _per_e((1, 1, MICRO_HID)),
            _per_e((1, MICRO_HID, DIM)),
            _per_e((1, 1, DIM)),
            _per_e((1, 1, DIM)),
            _per_e((1, 1, DIM)),
            _resident((1, DIM)),
            _resident((1, DIM)),
        ],
        out_specs=_resident((S, D)),
        out_shape=jax.ShapeDtypeStruct((S, D), jnp.float32),
        scratch_shapes=[pltpu.VMEM((S, D), jnp.float32)],
        compiler_params=pltpu.CompilerParams(
            dimension_semantics=("arbitrary", "arbitrary"),
        ),
    )(expert_output, micro_router_W, micro_router_b.reshape(1, -1),
      micro_W1, micro_b1.reshape(NUM_MICROS, 1, MICRO_HID), micro_W2,
      micro_b2.reshape(NUM_MICROS, 1, DIM),
      micro_ln_g.reshape(NUM_MICROS, 1, DIM),
      micro_ln_b.reshape(NUM_MICROS, 1, DIM),
      norm_g.reshape(1, -1), norm_b.reshape(1, -1))

    return out.reshape(B, S, D)


# speedup vs baseline: 2.9501x; 1.0350x over previous
import functools

import jax
import jax.numpy as jnp
from jax.experimental import pallas as pl
from jax.experimental.pallas import tpu as pltpu

DIM = 768
NUM_EXPERTS = 8
NUM_MICROS = 16
TOP_K = 2
TOP_K_MICROS = 8
EXPERT_DIM = 1536
MICRO_HID = DIM // 2
SEQ = 2048
TILE = 512
NUM_TILES = SEQ // TILE
EPS = 1e-5


def _gelu(v):
    return 0.5 * v * (1.0 + jax.lax.erf(v * 0.7071067811865476))


def _layer_norm(v, g, b):
    mu = jnp.mean(v, axis=-1, keepdims=True)
    var = jnp.mean((v - mu) ** 2, axis=-1, keepdims=True)
    return (v - mu) * jax.lax.rsqrt(var + EPS) * g + b


def _topk_mask_combine(probs, k):
    n = probs.shape[-1]
    work = probs
    thr = None
    sel_sum = jnp.zeros(probs.shape[:-1] + (1,), probs.dtype)
    for _ in range(k):
        thr = jnp.max(work, axis=-1, keepdims=True)
        sel_sum = sel_sum + thr
        work = jnp.where(work >= thr, -jnp.inf, work)
    mask = probs >= thr
    return jnp.where(mask, probs, 0.0) / (sel_sum + 1e-8)


def _expert_kernel(x_ref, rw_ref, rb_ref, w1_ref, b1_ref, w2_ref, b2_ref,
                   out_ref, cmb_ref):
    e = pl.program_id(0)
    t = pl.program_id(1)
    xt = x_ref[pl.ds(t * TILE, TILE), :]

    @pl.when(e == 0)
    def _router():
        logits = jnp.dot(xt, rw_ref[...], preferred_element_type=jnp.float32)
        logits = logits + rb_ref[...]
        probs = jax.nn.softmax(logits, axis=-1)
        cmb_ref[pl.ds(t * TILE, TILE), :] = _topk_mask_combine(probs, TOP_K)

    combine = cmb_ref[pl.ds(t * TILE, TILE), :]
    lane = jax.lax.broadcasted_iota(jnp.int32, combine.shape, 1)
    col = jnp.sum(jnp.where(lane == e, combine, 0.0), axis=-1, keepdims=True)

    h = jnp.dot(xt, w1_ref[0], preferred_element_type=jnp.float32)
    h = _gelu(h + b1_ref[0])
    eo = jnp.dot(h, w2_ref[0], preferred_element_type=jnp.float32)
    eo = (eo + b2_ref[0]) * col

    @pl.when(e == 0)
    def _init():
        out_ref[pl.ds(t * TILE, TILE), :] = eo

    @pl.when(e > 0)
    def _acc():
        out_ref[pl.ds(t * TILE, TILE), :] += eo


def _micro_kernel(eo_ref, rw_ref, rb_ref, w1_ref, b1_ref, w2_ref, b2_ref,
                  lng_ref, lnb_ref, ng_ref, nb_ref, out_ref, acc_ref,
                  cmb_ref):
    m = pl.program_id(0)
    t = pl.program_id(1)
    xt = eo_ref[pl.ds(t * TILE, TILE), :]

    @pl.when(m == 0)
    def _router():
        logits = jnp.dot(xt, rw_ref[...], preferred_element_type=jnp.float32)
        logits = logits + rb_ref[...]
        probs = jax.nn.softmax(logits, axis=-1)
        cmb_ref[pl.ds(t * TILE, TILE), :] = _topk_mask_combine(
            probs, TOP_K_MICROS)

    mcombine = cmb_ref[pl.ds(t * TILE, TILE), :]
    lane = jax.lax.broadcasted_iota(jnp.int32, mcombine.shape, 1)
    col = jnp.sum(jnp.where(lane == m, mcombine, 0.0), axis=-1, keepdims=True)

    mh = jnp.dot(xt, w1_ref[0], preferred_element_type=jnp.float32)
    mh = _gelu(mh + b1_ref[0])
    mf = jnp.dot(mh, w2_ref[0], preferred_element_type=jnp.float32)
    pre = xt + mf + b2_ref[0]
    mo = _layer_norm(pre, lng_ref[0], lnb_ref[0]) * col

    @pl.when(m == 0)
    def _init():
        acc_ref[pl.ds(t * TILE, TILE), :] = mo

    @pl.when(m > 0)
    def _acc():
        acc_ref[pl.ds(t * TILE, TILE), :] += mo

    @pl.when(m == NUM_MICROS - 1)
    def _final():
        combined = xt + 0.1 * acc_ref[pl.ds(t * TILE, TILE), :]
        out_ref[pl.ds(t * TILE, TILE), :] = _layer_norm(
            combined, ng_ref[...], nb_ref[...])


def _resident(shape):
    return pl.BlockSpec(shape, lambda *_: tuple(0 for _ in shape))


def _per_e(shape):
    return pl.BlockSpec(shape, lambda e, t: (e,) + tuple(0 for _ in shape[1:]))


@jax.jit
def kernel(x, router_W, router_b, expert_W1, expert_b1, expert_W2, expert_b2,
           micro_router_W, micro_router_b, micro_W1, micro_b1, micro_W2,
           micro_b2, micro_ln_g, micro_ln_b, norm_g, norm_b):
    B, S, D = x.shape
    xf = x.reshape(S, D)

    expert_output = pl.pallas_call(
        _expert_kernel,
        grid=(NUM_EXPERTS, NUM_TILES),
        in_specs=[
            _resident((S, D)),
            _resident((D, NUM_EXPERTS)),
            _resident((1, NUM_EXPERTS)),
            _per_e((1, DIM, EXPERT_DIM)),
            _per_e((1, 1, EXPERT_DIM)),
            _per_e((1, EXPERT_DIM, DIM)),
            _per_e((1, 1, DIM)),
        ],
        out_specs=_resident((S, D)),
        out_shape=jax.ShapeDtypeStruct((S, D), jnp.float32),
        scratch_shapes=[pltpu.VMEM((S, NUM_EXPERTS), jnp.float32)],
        compiler_params=pltpu.CompilerParams(
            dimension_semantics=("arbitrary", "arbitrary"),
        ),
    )(xf, router_W, router_b.reshape(1, -1), expert_W1,
      expert_b1.reshape(NUM_EXPERTS, 1, EXPERT_DIM), expert_W2,
      expert_b2.reshape(NUM_EXPERTS, 1, DIM))

    out = pl.pallas_call(
        _micro_kernel,
        grid=(NUM_MICROS, NUM_TILES),
        in_specs=[
            _resident((S, D)),
            _resident((D, NUM_MICROS)),
            _resident((1, NUM_MICROS)),
            _per_e((1, DIM, MICRO_HID)),
            _per_e((1, 1, MICRO_HID)),
            _per_e((1, MICRO_HID, DIM)),
            _per_e((1, 1, DIM)),
            _per_e((1, 1, DIM)),
            _per_e((1, 1, DIM)),
            _resident((1, DIM)),
            _resident((1, DIM)),
        ],
        out_specs=_resident((S, D)),
        out_shape=jax.ShapeDtypeStruct((S, D), jnp.float32),
        scratch_shapes=[pltpu.VMEM((S, D), jnp.float32),
                        pltpu.VMEM((S, NUM_MICROS), jnp.float32)],
        compiler_params=pltpu.CompilerParams(
            dimension_semantics=("arbitrary", "arbitrary"),
        ),
    )(expert_output, micro_router_W, micro_router_b.reshape(1, -1),
      micro_W1, micro_b1.reshape(NUM_MICROS, 1, MICRO_HID), micro_W2,
      micro_b2.reshape(NUM_MICROS, 1, DIM),
      micro_ln_g.reshape(NUM_MICROS, 1, DIM),
      micro_ln_b.reshape(NUM_MICROS, 1, DIM),
      norm_g.reshape(1, -1), norm_b.reshape(1, -1))

    return out.reshape(B, S, D)
